# baseline (speedup 1.0000x reference)
import functools

import jax
import jax.numpy as jnp
from jax.experimental import pallas as pl
from jax.experimental.pallas import tpu as pltpu


def _im2col(xp_ref, KW, N, TN):
    taps = [xp_ref[0, :, dt * N:dt * N + TN] for dt in range(KW)]
    return jnp.concatenate(taps, axis=0)


def _trend_body(wm_ref, bm_ref, wt_ref, bt_ref, xp_ref, tr_ref, z1_ref, st_ref,
                *, K, KW, C, N, T):
    TN = T * N
    im = _im2col(xp_ref, KW, N, TN)
    r = jnp.dot(wm_ref[...], im, preferred_element_type=jnp.float32, precision=jax.lax.Precision.DEFAULT) + bm_ref[...]
    z = jnp.zeros((C, TN), jnp.float32) + bt_ref[...]
    for k in range(1, K):
        r = jnp.concatenate(
            [jnp.dot(r[:, t * N:(t + 1) * N], tr_ref[0, t],
                     preferred_element_type=jnp.float32, precision=jax.lax.Precision.DEFAULT) for t in range(T)],
            axis=1)
        z = z + jnp.dot(wt_ref[:, (k - 1) * C:k * C], r,
                        preferred_element_type=jnp.float32, precision=jax.lax.Precision.DEFAULT)
    z1_ref[0] = z.astype(z1_ref.dtype)
    st_ref[0] = jnp.concatenate(
        [jnp.sum(z, axis=1, keepdims=True),
         jnp.sum(z * z, axis=1, keepdims=True)], axis=1)


def _out_body(wf_ref, bf_ref, w1_ref, w2_ref, bp_ref, lt_ref, xp_ref, z1_ref,
              o_ref, *, K, KW, C, N, T):
    TN = T * N
    im = _im2col(xp_ref, KW, N, TN)
    acc = jnp.dot(wf_ref[...], im, preferred_element_type=jnp.float32, precision=jax.lax.Precision.HIGHEST) + bf_ref[...]
    xm, x1 = acc[:C], acc[C:]
    z = (jnp.dot(w1_ref[...], z1_ref[0].astype(jnp.float32),
                 preferred_element_type=jnp.float32, precision=jax.lax.Precision.HIGHEST)
         + jnp.dot(w2_ref[:, :C], xm, preferred_element_type=jnp.float32, precision=jax.lax.Precision.HIGHEST)
         + bp_ref[...])
    for k in range(1, K):
        xk = jnp.concatenate(
            [jnp.dot(xm[:, t * N:(t + 1) * N], lt_ref[k - 1],
                     preferred_element_type=jnp.float32, precision=jax.lax.Precision.HIGHEST) for t in range(T)],
            axis=1)
        z = z + jnp.dot(w2_ref[:, k * C:(k + 1) * C], xk,
                        preferred_element_type=jnp.float32, precision=jax.lax.Precision.HIGHEST)
    filt = z[:C] + x1
    o_ref[0] = (filt * jax.nn.sigmoid(z[C:])).astype(o_ref.dtype)


def kernel(x, trend, adj, W_1, b_1, W_c1, b_c1, W_c5, b_c5, W_c7, b_c7,
           W_c9, b_c9, W_out, b_out, W_g, b_g, W_t, b_t, bn_gamma, bn_beta,
           W_f, b_f):
    with jax.default_matmul_precision("highest"):
        return _forward(x, trend, adj, W_1, b_1, W_c1, b_c1, W_c5, b_c5, W_c7,
                        b_c7, W_c9, b_c9, W_out, b_out, W_g, b_g, W_t, b_t,
                        bn_gamma, bn_beta, W_f, b_f)


def _forward(x, trend, adj, W_1, b_1, W_c1, b_c1, W_c5, b_c5, W_c7, b_c7,
             W_c9, b_c9, W_out, b_out, W_g, b_g, W_t, b_t, bn_gamma, bn_beta,
             W_f, b_f):
    B, c_in, N, T = x.shape
    C = W_1.shape[0]
    K = W_g.shape[1] // C
    KW = 9
    pad = KW // 2
    TN = T * N
    TpN = (T + 2 * pad) * N
    f32 = jnp.float32

    wstack = jnp.zeros((KW, c_in, 4 * C), f32)
    for q, W in enumerate((W_c1, W_c5, W_c7, W_c9)):
        kw = W.shape[-1]
        off = (KW - kw) // 2
        wstack = wstack.at[off:off + kw, :, q * C:(q + 1) * C].set(
            jnp.transpose(W[:, :, 0, :], (2, 1, 0)))
    wout = W_out[:, :, 0, 0].T
    w_m = (wstack.reshape(KW * c_in, 4 * C) @ wout).T
    b_m = (jnp.concatenate([b_c1, b_c5, b_c7, b_c9], 0) @ wout + b_out)
    w_one = jnp.zeros((KW, c_in, C), f32).at[pad].set(
        W_1[:, :, 0, 0].T).reshape(KW * c_in, C).T
    w_front = jnp.concatenate([w_m, w_one], axis=0)
    b_front = jnp.concatenate([b_m, b_1], 0).reshape(2 * C, 1)

    xt = jnp.transpose(x, (0, 1, 3, 2))
    xpad = jnp.pad(xt, ((0, 0), (0, 0), (pad, pad), (0, 0))
                   ).reshape(B, c_in, TpN)

    z1, stats = pl.pallas_call(
        functools.partial(_trend_body, K=K, KW=KW, C=C, N=N, T=T),
        out_shape=(jax.ShapeDtypeStruct((B, C, TN), x.dtype),
                   jax.ShapeDtypeStruct((B, C, 2), f32)),
        grid=(B,),
        in_specs=[
            pl.BlockSpec((C, KW * c_in), lambda b: (0, 0)),
            pl.BlockSpec((C, 1), lambda b: (0, 0)),
            pl.BlockSpec((C, (K - 1) * C), lambda b: (0, 0)),
            pl.BlockSpec((C, 1), lambda b: (0, 0)),
            pl.BlockSpec((1, c_in, TpN), lambda b: (b, 0, 0)),
            pl.BlockSpec((1, T, N, N), lambda b: (b, 0, 0, 0)),
        ],
        out_specs=(pl.BlockSpec((1, C, TN), lambda b: (b, 0, 0)),
                   pl.BlockSpec((1, C, 2), lambda b: (b, 0, 0))),
        compiler_params=pltpu.CompilerParams(
            dimension_semantics=("parallel",),
            vmem_limit_bytes=64 * 1024 * 1024),
    )(w_m, b_m.reshape(C, 1), W_t[:, :, 0, 0], b_t.reshape(C, 1), xpad, trend)

    sums = stats.sum(axis=0)
    cnt = jnp.float32(B * TN)
    mean = sums[:, 0] / cnt
    var = sums[:, 1] / cnt - mean * mean
    scale = bn_gamma * jax.lax.rsqrt(var + 1e-5)
    shift = bn_beta - mean * scale
    A_f = W_f[:, :, 0, 0]
    A_f1, A_f2 = A_f[:, :C], A_f[:, C:]
    A_g = (W_g[:, :, 0, 0].reshape(2 * C, C, K)
           .transpose(0, 2, 1).reshape(2 * C, K * C))
    w1p = A_f1 * scale[None, :]
    w2p = A_f2 @ A_g
    bp = (A_f1 @ shift + A_f2 @ b_g + b_f).reshape(2 * C, 1)

    L0, L1 = jnp.eye(N, dtype=f32), adj
    lts = [L1.T]
    for _ in range(2, K):
        L2 = 2.0 * adj @ L1 - L0
        L0, L1 = L1, L2
        lts.append(L2.T)
    lt = jnp.stack(lts, axis=0)

    out = pl.pallas_call(
        functools.partial(_out_body, K=K, KW=KW, C=C, N=N, T=T),
        out_shape=jax.ShapeDtypeStruct((B, C, TN), x.dtype),
        grid=(B,),
        in_specs=[
            pl.BlockSpec((2 * C, KW * c_in), lambda b: (0, 0)),
            pl.BlockSpec((2 * C, 1), lambda b: (0, 0)),
            pl.BlockSpec((2 * C, C), lambda b: (0, 0)),
            pl.BlockSpec((2 * C, K * C), lambda b: (0, 0)),
            pl.BlockSpec((2 * C, 1), lambda b: (0, 0)),
            pl.BlockSpec((K - 1, N, N), lambda b: (0, 0, 0)),
            pl.BlockSpec((1, c_in, TpN), lambda b: (b, 0, 0)),
            pl.BlockSpec((1, C, TN), lambda b: (b, 0, 0)),
        ],
        out_specs=pl.BlockSpec((1, C, TN), lambda b: (b, 0, 0)),
        compiler_params=pltpu.CompilerParams(
            dimension_semantics=("parallel",),
            vmem_limit_bytes=64 * 1024 * 1024),
    )(w_front, b_front, w1p, w2p, bp, lt, xpad, z1)

    return jnp.transpose(out.reshape(B, C, T, N), (0, 1, 3, 2))


# speedup vs baseline: 1.8770x; 1.8770x over previous
import functools

import jax
import jax.numpy as jnp
from jax.experimental import pallas as pl
from jax.experimental.pallas import tpu as pltpu


def _im2col(xp_ref, KW, N, TN):
    taps = [xp_ref[0, :, dt * N:dt * N + TN] for dt in range(KW)]
    return jnp.concatenate(taps, axis=0)


def _trend_body(wm_ref, bm_ref, wt_ref, bt_ref, xp_ref, tr_ref, z1_ref, st_ref,
                *, K, KW, C, N, T):
    TN = T * N
    im = _im2col(xp_ref, KW, N, TN)
    r = jnp.dot(wm_ref[...], im, preferred_element_type=jnp.float32, precision=jax.lax.Precision.DEFAULT) + bm_ref[...]
    z = jnp.zeros((C, TN), jnp.float32) + bt_ref[...]
    for k in range(1, K):
        r = jnp.concatenate(
            [jnp.dot(r[:, t * N:(t + 1) * N], tr_ref[0, t],
                     preferred_element_type=jnp.float32, precision=jax.lax.Precision.DEFAULT) for t in range(T)],
            axis=1)
        z = z + jnp.dot(wt_ref[:, (k - 1) * C:k * C], r,
                        preferred_element_type=jnp.float32, precision=jax.lax.Precision.DEFAULT)
    z1_ref[0] = z.astype(z1_ref.dtype)
    st_ref[0] = jnp.concatenate(
        [jnp.sum(z, axis=1, keepdims=True),
         jnp.sum(z * z, axis=1, keepdims=True)], axis=1)


def _out_body(wf_ref, bf_ref, w1_ref, w2_ref, bp_ref, lt_ref, xp_ref, z1_ref,
              o_ref, *, K, KW, C, N, T):
    TN = T * N
    im = _im2col(xp_ref, KW, N, TN)
    acc = jnp.dot(wf_ref[...], im, preferred_element_type=jnp.float32, precision=jax.lax.Precision.HIGHEST) + bf_ref[...]
    xm, x1 = acc[:C], acc[C:]
    z = (jnp.dot(w1_ref[...], z1_ref[0].astype(jnp.float32),
                 preferred_element_type=jnp.float32, precision=jax.lax.Precision.DEFAULT)
         + jnp.dot(w2_ref[:, :C], xm, preferred_element_type=jnp.float32, precision=jax.lax.Precision.DEFAULT)
         + bp_ref[...])
    for k in range(1, K):
        xk = jnp.concatenate(
            [jnp.dot(xm[:, t * N:(t + 1) * N], lt_ref[k - 1],
                     preferred_element_type=jnp.float32, precision=jax.lax.Precision.DEFAULT) for t in range(T)],
            axis=1)
        z = z + jnp.dot(w2_ref[:, k * C:(k + 1) * C], xk,
                        preferred_element_type=jnp.float32, precision=jax.lax.Precision.DEFAULT)
    filt = z[:C] + x1
    o_ref[0] = (filt * jax.nn.sigmoid(z[C:])).astype(o_ref.dtype)


def kernel(x, trend, adj, W_1, b_1, W_c1, b_c1, W_c5, b_c5, W_c7, b_c7,
           W_c9, b_c9, W_out, b_out, W_g, b_g, W_t, b_t, bn_gamma, bn_beta,
           W_f, b_f):
    with jax.default_matmul_precision("highest"):
        return _forward(x, trend, adj, W_1, b_1, W_c1, b_c1, W_c5, b_c5, W_c7,
                        b_c7, W_c9, b_c9, W_out, b_out, W_g, b_g, W_t, b_t,
                        bn_gamma, bn_beta, W_f, b_f)


def _forward(x, trend, adj, W_1, b_1, W_c1, b_c1, W_c5, b_c5, W_c7, b_c7,
             W_c9, b_c9, W_out, b_out, W_g, b_g, W_t, b_t, bn_gamma, bn_beta,
             W_f, b_f):
    B, c_in, N, T = x.shape
    C = W_1.shape[0]
    K = W_g.shape[1] // C
    KW = 9
    pad = KW // 2
    TN = T * N
    TpN = (T + 2 * pad) * N
    f32 = jnp.float32

    wstack = jnp.zeros((KW, c_in, 4 * C), f32)
    for q, W in enumerate((W_c1, W_c5, W_c7, W_c9)):
        kw = W.shape[-1]
        off = (KW - kw) // 2
        wstack = wstack.at[off:off + kw, :, q * C:(q + 1) * C].set(
            jnp.transpose(W[:, :, 0, :], (2, 1, 0)))
    wout = W_out[:, :, 0, 0].T
    w_m = (wstack.reshape(KW * c_in, 4 * C) @ wout).T
    b_m = (jnp.concatenate([b_c1, b_c5, b_c7, b_c9], 0) @ wout + b_out)
    w_one = jnp.zeros((KW, c_in, C), f32).at[pad].set(
        W_1[:, :, 0, 0].T).reshape(KW * c_in, C).T
    w_front = jnp.concatenate([w_m, w_one], axis=0)
    b_front = jnp.concatenate([b_m, b_1], 0).reshape(2 * C, 1)

    xt = jnp.transpose(x, (0, 1, 3, 2))
    xpad = jnp.pad(xt, ((0, 0), (0, 0), (pad, pad), (0, 0))
                   ).reshape(B, c_in, TpN)

    z1, stats = pl.pallas_call(
        functools.partial(_trend_body, K=K, KW=KW, C=C, N=N, T=T),
        out_shape=(jax.ShapeDtypeStruct((B, C, TN), x.dtype),
                   jax.ShapeDtypeStruct((B, C, 2), f32)),
        grid=(B,),
        in_specs=[
            pl.BlockSpec((C, KW * c_in), lambda b: (0, 0)),
            pl.BlockSpec((C, 1), lambda b: (0, 0)),
            pl.BlockSpec((C, (K - 1) * C), lambda b: (0, 0)),
            pl.BlockSpec((C, 1), lambda b: (0, 0)),
            pl.BlockSpec((1, c_in, TpN), lambda b: (b, 0, 0)),
            pl.BlockSpec((1, T, N, N), lambda b: (b, 0, 0, 0)),
        ],
        out_specs=(pl.BlockSpec((1, C, TN), lambda b: (b, 0, 0)),
                   pl.BlockSpec((1, C, 2), lambda b: (b, 0, 0))),
        compiler_params=pltpu.CompilerParams(
            dimension_semantics=("parallel",),
            vmem_limit_bytes=64 * 1024 * 1024),
    )(w_m, b_m.reshape(C, 1), W_t[:, :, 0, 0], b_t.reshape(C, 1), xpad, trend)

    sums = stats.sum(axis=0)
    cnt = jnp.float32(B * TN)
    mean = sums[:, 0] / cnt
    var = sums[:, 1] / cnt - mean * mean
    scale = bn_gamma * jax.lax.rsqrt(var + 1e-5)
    shift = bn_beta - mean * scale
    A_f = W_f[:, :, 0, 0]
    A_f1, A_f2 = A_f[:, :C], A_f[:, C:]
    A_g = (W_g[:, :, 0, 0].reshape(2 * C, C, K)
           .transpose(0, 2, 1).reshape(2 * C, K * C))
    w1p = A_f1 * scale[None, :]
    w2p = A_f2 @ A_g
    bp = (A_f1 @ shift + A_f2 @ b_g + b_f).reshape(2 * C, 1)

    L0, L1 = jnp.eye(N, dtype=f32), adj
    lts = [L1.T]
    for _ in range(2, K):
        L2 = 2.0 * adj @ L1 - L0
        L0, L1 = L1, L2
        lts.append(L2.T)
    lt = jnp.stack(lts, axis=0)

    out = pl.pallas_call(
        functools.partial(_out_body, K=K, KW=KW, C=C, N=N, T=T),
        out_shape=jax.ShapeDtypeStruct((B, C, TN), x.dtype),
        grid=(B,),
        in_specs=[
            pl.BlockSpec((2 * C, KW * c_in), lambda b: (0, 0)),
            pl.BlockSpec((2 * C, 1), lambda b: (0, 0)),
            pl.BlockSpec((2 * C, C), lambda b: (0, 0)),
            pl.BlockSpec((2 * C, K * C), lambda b: (0, 0)),
            pl.BlockSpec((2 * C, 1), lambda b: (0, 0)),
            pl.BlockSpec((K - 1, N, N), lambda b: (0, 0, 0)),
            pl.BlockSpec((1, c_in, TpN), lambda b: (b, 0, 0)),
            pl.BlockSpec((1, C, TN), lambda b: (b, 0, 0)),
        ],
        out_specs=pl.BlockSpec((1, C, TN), lambda b: (b, 0, 0)),
        compiler_params=pltpu.CompilerParams(
            dimension_semantics=("parallel",),
            vmem_limit_bytes=64 * 1024 * 1024),
    )(w_front, b_front, w1p, w2p, bp, lt, xpad, z1)

    return jnp.transpose(out.reshape(B, C, T, N), (0, 1, 3, 2))


# speedup vs baseline: 2.0562x; 1.0955x over previous
import functools

import jax
import jax.numpy as jnp
from jax.experimental import pallas as pl
from jax.experimental.pallas import tpu as pltpu


def _im2col(xp_ref, KW, N, TN):
    taps = [xp_ref[0, :, dt * N:dt * N + TN] for dt in range(KW)]
    return jnp.concatenate(taps, axis=0)


def _dot3(a, b):
    ah = a.astype(jnp.bfloat16)
    al = (a - ah.astype(jnp.float32)).astype(jnp.bfloat16)
    bh = b.astype(jnp.bfloat16)
    bl = (b - bh.astype(jnp.float32)).astype(jnp.bfloat16)

    def d(p, q):
        return jnp.dot(p, q, preferred_element_type=jnp.float32,
                       precision=jax.lax.Precision.DEFAULT)

    return d(ah, bh) + d(ah, bl) + d(al, bh)


def _trend_body(wm_ref, bm_ref, wt_ref, bt_ref, xp_ref, tr_ref, z1_ref, st_ref,
                *, K, KW, C, N, T):
    TN = T * N
    im = _im2col(xp_ref, KW, N, TN)
    r = jnp.dot(wm_ref[...], im, preferred_element_type=jnp.float32, precision=jax.lax.Precision.DEFAULT) + bm_ref[...]
    z = jnp.zeros((C, TN), jnp.float32) + bt_ref[...]
    for k in range(1, K):
        r = jnp.concatenate(
            [jnp.dot(r[:, t * N:(t + 1) * N], tr_ref[0, t],
                     preferred_element_type=jnp.float32, precision=jax.lax.Precision.DEFAULT) for t in range(T)],
            axis=1)
        z = z + jnp.dot(wt_ref[:, (k - 1) * C:k * C], r,
                        preferred_element_type=jnp.float32, precision=jax.lax.Precision.DEFAULT)
    z1_ref[0] = z.astype(z1_ref.dtype)
    st_ref[0] = jnp.concatenate(
        [jnp.sum(z, axis=1, keepdims=True),
         jnp.sum(z * z, axis=1, keepdims=True)], axis=1)


def _out_body(wf_ref, bf_ref, w1_ref, w2_ref, bp_ref, lt_ref, xp_ref, z1_ref,
              o_ref, *, K, KW, C, N, T):
    TN = T * N
    im = _im2col(xp_ref, KW, N, TN)
    acc = _dot3(wf_ref[...], im) + bf_ref[...]
    xm, x1 = acc[:C], acc[C:]
    z = (jnp.dot(w1_ref[...], z1_ref[0].astype(jnp.float32),
                 preferred_element_type=jnp.float32, precision=jax.lax.Precision.DEFAULT)
         + jnp.dot(w2_ref[:, :C], xm, preferred_element_type=jnp.float32, precision=jax.lax.Precision.DEFAULT)
         + bp_ref[...])
    for k in range(1, K):
        xk = jnp.concatenate(
            [jnp.dot(xm[:, t * N:(t + 1) * N], lt_ref[k - 1],
                     preferred_element_type=jnp.float32, precision=jax.lax.Precision.DEFAULT) for t in range(T)],
            axis=1)
        z = z + jnp.dot(w2_ref[:, k * C:(k + 1) * C], xk,
                        preferred_element_type=jnp.float32, precision=jax.lax.Precision.DEFAULT)
    filt = z[:C] + x1
    o_ref[0] = (filt * jax.nn.sigmoid(z[C:])).astype(o_ref.dtype)


def kernel(x, trend, adj, W_1, b_1, W_c1, b_c1, W_c5, b_c5, W_c7, b_c7,
           W_c9, b_c9, W_out, b_out, W_g, b_g, W_t, b_t, bn_gamma, bn_beta,
           W_f, b_f):
    with jax.default_matmul_precision("highest"):
        return _forward(x, trend, adj, W_1, b_1, W_c1, b_c1, W_c5, b_c5, W_c7,
                        b_c7, W_c9, b_c9, W_out, b_out, W_g, b_g, W_t, b_t,
                        bn_gamma, bn_beta, W_f, b_f)


def _forward(x, trend, adj, W_1, b_1, W_c1, b_c1, W_c5, b_c5, W_c7, b_c7,
             W_c9, b_c9, W_out, b_out, W_g, b_g, W_t, b_t, bn_gamma, bn_beta,
             W_f, b_f):
    B, c_in, N, T = x.shape
    C = W_1.shape[0]
    K = W_g.shape[1] // C
    KW = 9
    pad = KW // 2
    TN = T * N
    TpN = (T + 2 * pad) * N
    f32 = jnp.float32

    wstack = jnp.zeros((KW, c_in, 4 * C), f32)
    for q, W in enumerate((W_c1, W_c5, W_c7, W_c9)):
        kw = W.shape[-1]
        off = (KW - kw) // 2
        wstack = wstack.at[off:off + kw, :, q * C:(q + 1) * C].set(
            jnp.transpose(W[:, :, 0, :], (2, 1, 0)))
    wout = W_out[:, :, 0, 0].T
    w_m = (wstack.reshape(KW * c_in, 4 * C) @ wout).T
    b_m = (jnp.concatenate([b_c1, b_c5, b_c7, b_c9], 0) @ wout + b_out)
    w_one = jnp.zeros((KW, c_in, C), f32).at[pad].set(
        W_1[:, :, 0, 0].T).reshape(KW * c_in, C).T
    w_front = jnp.concatenate([w_m, w_one], axis=0)
    b_front = jnp.concatenate([b_m, b_1], 0).reshape(2 * C, 1)

    xt = jnp.transpose(x, (0, 1, 3, 2))
    xpad = jnp.pad(xt, ((0, 0), (0, 0), (pad, pad), (0, 0))
                   ).reshape(B, c_in, TpN)

    z1, stats = pl.pallas_call(
        functools.partial(_trend_body, K=K, KW=KW, C=C, N=N, T=T),
        out_shape=(jax.ShapeDtypeStruct((B, C, TN), x.dtype),
                   jax.ShapeDtypeStruct((B, C, 2), f32)),
        grid=(B,),
        in_specs=[
            pl.BlockSpec((C, KW * c_in), lambda b: (0, 0)),
            pl.BlockSpec((C, 1), lambda b: (0, 0)),
            pl.BlockSpec((C, (K - 1) * C), lambda b: (0, 0)),
            pl.BlockSpec((C, 1), lambda b: (0, 0)),
            pl.BlockSpec((1, c_in, TpN), lambda b: (b, 0, 0)),
            pl.BlockSpec((1, T, N, N), lambda b: (b, 0, 0, 0)),
        ],
        out_specs=(pl.BlockSpec((1, C, TN), lambda b: (b, 0, 0)),
                   pl.BlockSpec((1, C, 2), lambda b: (b, 0, 0))),
        compiler_params=pltpu.CompilerParams(
            dimension_semantics=("parallel",),
            vmem_limit_bytes=64 * 1024 * 1024),
    )(w_m, b_m.reshape(C, 1), W_t[:, :, 0, 0], b_t.reshape(C, 1), xpad, trend)

    sums = stats.sum(axis=0)
    cnt = jnp.float32(B * TN)
    mean = sums[:, 0] / cnt
    var = sums[:, 1] / cnt - mean * mean
    scale = bn_gamma * jax.lax.rsqrt(var + 1e-5)
    shift = bn_beta - mean * scale
    A_f = W_f[:, :, 0, 0]
    A_f1, A_f2 = A_f[:, :C], A_f[:, C:]
    A_g = (W_g[:, :, 0, 0].reshape(2 * C, C, K)
           .transpose(0, 2, 1).reshape(2 * C, K * C))
    w1p = A_f1 * scale[None, :]
    w2p = A_f2 @ A_g
    bp = (A_f1 @ shift + A_f2 @ b_g + b_f).reshape(2 * C, 1)

    L0, L1 = jnp.eye(N, dtype=f32), adj
    lts = [L1.T]
    for _ in range(2, K):
        L2 = 2.0 * adj @ L1 - L0
        L0, L1 = L1, L2
        lts.append(L2.T)
    lt = jnp.stack(lts, axis=0)

    out = pl.pallas_call(
        functools.partial(_out_body, K=K, KW=KW, C=C, N=N, T=T),
        out_shape=jax.ShapeDtypeStruct((B, C, TN), x.dtype),
        grid=(B,),
        in_specs=[
            pl.BlockSpec((2 * C, KW * c_in), lambda b: (0, 0)),
            pl.BlockSpec((2 * C, 1), lambda b: (0, 0)),
            pl.BlockSpec((2 * C, C), lambda b: (0, 0)),
            pl.BlockSpec((2 * C, K * C), lambda b: (0, 0)),
            pl.BlockSpec((2 * C, 1), lambda b: (0, 0)),
            pl.BlockSpec((K - 1, N, N), lambda b: (0, 0, 0)),
            pl.BlockSpec((1, c_in, TpN), lambda b: (b, 0, 0)),
            pl.BlockSpec((1, C, TN), lambda b: (b, 0, 0)),
        ],
        out_specs=pl.BlockSpec((1, C, TN), lambda b: (b, 0, 0)),
        compiler_params=pltpu.CompilerParams(
            dimension_semantics=("parallel",),
            vmem_limit_bytes=64 * 1024 * 1024),
    )(w_front, b_front, w1p, w2p, bp, lt, xpad, z1)

    return jnp.transpose(out.reshape(B, C, T, N), (0, 1, 3, 2))


# speedup vs baseline: 2.3068x; 1.1219x over previous
import functools

import jax
import jax.numpy as jnp
from jax.experimental import pallas as pl
from jax.experimental.pallas import tpu as pltpu


def _im2col(xp_ref, KW, N, TN):
    taps = [xp_ref[0, :, dt * N:dt * N + TN] for dt in range(KW)]
    return jnp.concatenate(taps, axis=0)


def _dot(a, b):
    return jnp.dot(a, b, preferred_element_type=jnp.float32,
                   precision=jax.lax.Precision.DEFAULT)


def _dot3(a, b):
    ah = a.astype(jnp.bfloat16)
    al = (a - ah.astype(jnp.float32)).astype(jnp.bfloat16)
    bh = b.astype(jnp.bfloat16)
    bl = (b - bh.astype(jnp.float32)).astype(jnp.bfloat16)
    return _dot(ah, bh) + _dot(ah, bl) + _dot(al, bh)


def _trend_body(wm_ref, bm_ref, wt_ref, bt_ref, xp_ref, tr_ref, z1_ref, st_ref,
                *, K, KW, C, N, T):
    TN = T * N
    im = _im2col(xp_ref, KW, N, TN)
    r = _dot(wm_ref[...], im) + bm_ref[...]
    rs = []
    for _ in range(1, K):
        r = jnp.concatenate(
            [_dot(r[:, t * N:(t + 1) * N], tr_ref[0, t]) for t in range(T)],
            axis=1)
        rs.append(r)
    z = _dot(wt_ref[...], jnp.concatenate(rs, axis=0)) + bt_ref[...]
    z1_ref[0] = z.astype(z1_ref.dtype)
    st_ref[0] = jnp.concatenate(
        [jnp.sum(z, axis=1, keepdims=True),
         jnp.sum(z * z, axis=1, keepdims=True)], axis=1)


def _out_body(wf_ref, bf_ref, w12_ref, w2r_ref, bp_ref, ltc_ref, xp_ref,
              z1_ref, o_ref, *, K, KW, C, N, T):
    TN = T * N
    im = _im2col(xp_ref, KW, N, TN)
    acc = _dot3(wf_ref[...], im) + bf_ref[...]
    xm, x1 = acc[:C], acc[C:]
    parts = [_dot(xm[:, t * N:(t + 1) * N], ltc_ref[...]) for t in range(T)]
    xks = jnp.concatenate(
        [jnp.concatenate([p[:, (k - 1) * N:k * N] for p in parts], axis=1)
         for k in range(1, K)], axis=0)
    u = jnp.concatenate([z1_ref[0].astype(jnp.float32), xm], axis=0)
    z = _dot(w12_ref[...], u) + _dot(w2r_ref[...], xks) + bp_ref[...]
    filt = z[:C] + x1
    o_ref[0] = (filt * jax.nn.sigmoid(z[C:])).astype(o_ref.dtype)


def kernel(x, trend, adj, W_1, b_1, W_c1, b_c1, W_c5, b_c5, W_c7, b_c7,
           W_c9, b_c9, W_out, b_out, W_g, b_g, W_t, b_t, bn_gamma, bn_beta,
           W_f, b_f):
    with jax.default_matmul_precision("highest"):
        return _forward(x, trend, adj, W_1, b_1, W_c1, b_c1, W_c5, b_c5, W_c7,
                        b_c7, W_c9, b_c9, W_out, b_out, W_g, b_g, W_t, b_t,
                        bn_gamma, bn_beta, W_f, b_f)


def _forward(x, trend, adj, W_1, b_1, W_c1, b_c1, W_c5, b_c5, W_c7, b_c7,
             W_c9, b_c9, W_out, b_out, W_g, b_g, W_t, b_t, bn_gamma, bn_beta,
             W_f, b_f):
    B, c_in, N, T = x.shape
    C = W_1.shape[0]
    K = W_g.shape[1] // C
    KW = 9
    pad = KW // 2
    TN = T * N
    TpN = (T + 2 * pad) * N
    f32 = jnp.float32

    wstack = jnp.zeros((KW, c_in, 4 * C), f32)
    for q, W in enumerate((W_c1, W_c5, W_c7, W_c9)):
        kw = W.shape[-1]
        off = (KW - kw) // 2
        wstack = wstack.at[off:off + kw, :, q * C:(q + 1) * C].set(
            jnp.transpose(W[:, :, 0, :], (2, 1, 0)))
    wout = W_out[:, :, 0, 0].T
    w_m = (wstack.reshape(KW * c_in, 4 * C) @ wout).T
    b_m = (jnp.concatenate([b_c1, b_c5, b_c7, b_c9], 0) @ wout + b_out)
    w_one = jnp.zeros((KW, c_in, C), f32).at[pad].set(
        W_1[:, :, 0, 0].T).reshape(KW * c_in, C).T
    w_front = jnp.concatenate([w_m, w_one], axis=0)
    b_front = jnp.concatenate([b_m, b_1], 0).reshape(2 * C, 1)

    xt = jnp.transpose(x, (0, 1, 3, 2))
    xpad = jnp.pad(xt, ((0, 0), (0, 0), (pad, pad), (0, 0))
                   ).reshape(B, c_in, TpN)

    z1, stats = pl.pallas_call(
        functools.partial(_trend_body, K=K, KW=KW, C=C, N=N, T=T),
        out_shape=(jax.ShapeDtypeStruct((B, C, TN), jnp.bfloat16),
                   jax.ShapeDtypeStruct((B, C, 2), f32)),
        grid=(B,),
        in_specs=[
            pl.BlockSpec((C, KW * c_in), lambda b: (0, 0)),
            pl.BlockSpec((C, 1), lambda b: (0, 0)),
            pl.BlockSpec((C, (K - 1) * C), lambda b: (0, 0)),
            pl.BlockSpec((C, 1), lambda b: (0, 0)),
            pl.BlockSpec((1, c_in, TpN), lambda b: (b, 0, 0)),
            pl.BlockSpec((1, T, N, N), lambda b: (b, 0, 0, 0)),
        ],
        out_specs=(pl.BlockSpec((1, C, TN), lambda b: (b, 0, 0)),
                   pl.BlockSpec((1, C, 2), lambda b: (b, 0, 0))),
        compiler_params=pltpu.CompilerParams(
            dimension_semantics=("parallel",),
            vmem_limit_bytes=64 * 1024 * 1024),
    )(w_m, b_m.reshape(C, 1), W_t[:, :, 0, 0], b_t.reshape(C, 1), xpad, trend)

    sums = stats.sum(axis=0)
    cnt = jnp.float32(B * TN)
    mean = sums[:, 0] / cnt
    var = sums[:, 1] / cnt - mean * mean
    scale = bn_gamma * jax.lax.rsqrt(var + 1e-5)
    shift = bn_beta - mean * scale
    A_f = W_f[:, :, 0, 0]
    A_f1, A_f2 = A_f[:, :C], A_f[:, C:]
    A_g = (W_g[:, :, 0, 0].reshape(2 * C, C, K)
           .transpose(0, 2, 1).reshape(2 * C, K * C))
    w1p = A_f1 * scale[None, :]
    w2p = A_f2 @ A_g
    bp = (A_f1 @ shift + A_f2 @ b_g + b_f).reshape(2 * C, 1)
    w12 = jnp.concatenate([w1p, w2p[:, :C]], axis=1)
    w2r = w2p[:, C:]

    L0, L1 = jnp.eye(N, dtype=f32), adj
    lts = [L1.T]
    for _ in range(2, K):
        L2 = 2.0 * adj @ L1 - L0
        L0, L1 = L1, L2
        lts.append(L2.T)
    ltc = jnp.concatenate(lts, axis=1)

    out = pl.pallas_call(
        functools.partial(_out_body, K=K, KW=KW, C=C, N=N, T=T),
        out_shape=jax.ShapeDtypeStruct((B, C, TN), x.dtype),
        grid=(B,),
        in_specs=[
            pl.BlockSpec((2 * C, KW * c_in), lambda b: (0, 0)),
            pl.BlockSpec((2 * C, 1), lambda b: (0, 0)),
            pl.BlockSpec((2 * C, 2 * C), lambda b: (0, 0)),
            pl.BlockSpec((2 * C, (K - 1) * C), lambda b: (0, 0)),
            pl.BlockSpec((2 * C, 1), lambda b: (0, 0)),
            pl.BlockSpec((N, (K - 1) * N), lambda b: (0, 0)),
            pl.BlockSpec((1, c_in, TpN), lambda b: (b, 0, 0)),
            pl.BlockSpec((1, C, TN), lambda b: (b, 0, 0)),
        ],
        out_specs=pl.BlockSpec((1, C, TN), lambda b: (b, 0, 0)),
        compiler_params=pltpu.CompilerParams(
            dimension_semantics=("parallel",),
            vmem_limit_bytes=64 * 1024 * 1024),
    )(w_front, b_front, w12, w2r, bp, ltc, xpad, z1)

    return jnp.transpose(out.reshape(B, C, T, N), (0, 1, 3, 2))
